# baseline (speedup 1.0000x reference)
import functools

import jax
import jax.numpy as jnp
from jax import lax
from jax.experimental import pallas as pl
from jax.experimental.pallas import tpu as pltpu
from jax.experimental.pallas import tpu_sc as plsc

N = 10000
E = 320000
D = 128
L = 10

NC = 2
NS = 16
NW = NC * NS
CHUNK = 80
NCHUNKS = 128
EPT = CHUNK * NCHUNKS
E_PAD = EPT * NW
N_PAD = 10240
SLAB = N_PAD // NS
LAST = N - (NS - 1) * SLAB
RD = 2
NG = NCHUNKS // RD
DEG_W = D

_MESH = plsc.VectorSubcoreMesh(
    core_axis_name="c", subcore_axis_name="s", num_cores=NC, num_subcores=NS
)


def _fill(buf, val, width=D):
    vec = jnp.full((16,), val, jnp.float32)

    def body(r, _):
        for k in range(width // 16):
            buf[r, pl.ds(k * 16, 16)] = vec
        return 0

    lax.fori_loop(0, CHUNK, body, 0)


def _zero_slab(zbuf, acc_sh, s, width=D):
    _fill(zbuf, 0.0, width)
    slab = pl.multiple_of(s * SLAB, CHUNK)
    for k in range(SLAB // CHUNK):
        pltpu.sync_copy(zbuf, acc_sh.at[pl.ds(slab + k * CHUNK, CHUNK)])


def _copy_out(acc_sh, out_hbm, c, s):
    start = pl.multiple_of(s * SLAB, CHUNK)

    @pl.when(s < NS - 1)
    def _():
        pltpu.sync_copy(acc_sh.at[pl.ds(start, SLAB)],
                        out_hbm.at[c, pl.ds(start, SLAB)])

    @pl.when(s == NS - 1)
    def _():
        pltpu.sync_copy(acc_sh.at[pl.ds(start, LAST)],
                        out_hbm.at[c, pl.ds(start, LAST)])


@functools.partial(
    pl.kernel,
    out_type=jax.ShapeDtypeStruct((NC, N, D), jnp.float32),
    mesh=_MESH,
    scratch_types=[
        [pltpu.VMEM((CHUNK, 2 * D), jnp.float32)] * RD,
        [pltpu.VMEM((CHUNK,), jnp.int32)] * RD,
        [pltpu.VMEM((CHUNK,), jnp.int32)] * RD,
        pltpu.VMEM_SHARED((N_PAD, D), jnp.float32),
        [pltpu.SemaphoreType.DMA] * RD,
        [pltpu.SemaphoreType.DMA] * RD,
        [pltpu.SemaphoreType.DMA] * RD,
        [pltpu.SemaphoreType.DMA] * RD,
    ],
)
def _sc_agg(h_hbm, idx_hbm, out_hbm,
            rows, srcb, dstb, acc_sh, xsems, dsems, gsems, ssems):
    c = lax.axis_index("c")
    s = lax.axis_index("s")
    wid = s * NC + c

    for b in range(RD):
        pltpu.async_copy(idx_hbm.at[wid, b, 0], srcb[b], xsems[b])
        pltpu.async_copy(idx_hbm.at[wid, b, 1], dstb[b], dsems[b])
    plsc.subcore_barrier()

    for b in range(RD):
        pltpu.make_async_copy(idx_hbm.at[wid, 0, 0], srcb[b], xsems[b]).wait()
        pltpu.async_copy(h_hbm.at[srcb[b]], rows[b], gsems[b])

    def group(g, _):
        for b in range(RD):
            ci = RD * g + b
            pltpu.make_async_copy(h_hbm.at[srcb[b]], rows[b],
                                  gsems[b]).wait()
            pltpu.async_copy(idx_hbm.at[wid, ci + RD, 0], srcb[b], xsems[b])
        for b in range(RD):
            pltpu.make_async_copy(idx_hbm.at[wid, 0, 0], srcb[b],
                                  xsems[b]).wait()
            pltpu.async_copy(h_hbm.at[srcb[b]], rows[b], gsems[b])
        return 0

    lax.fori_loop(0, NG - 1, group, 0)

    for b in range(RD):
        pltpu.make_async_copy(h_hbm.at[srcb[b]], rows[b], gsems[b]).wait()

    plsc.subcore_barrier()
    _copy_out(acc_sh, out_hbm, c, s)


@functools.partial(
    pl.kernel,
    out_type=jax.ShapeDtypeStruct((NC, N, DEG_W), jnp.float32),
    mesh=_MESH,
    scratch_types=[
        pltpu.VMEM((CHUNK, DEG_W), jnp.float32),
        pltpu.VMEM((CHUNK,), jnp.int32),
        pltpu.VMEM_SHARED((N_PAD, DEG_W), jnp.float32),
        pltpu.SemaphoreType.DMA,
    ],
)
def _sc_deg(idx_hbm, out_hbm, ones_v, dst0, acc_sh, sem):
    c = lax.axis_index("c")
    s = lax.axis_index("s")
    wid = s * NC + c

    _zero_slab(ones_v, acc_sh, s, DEG_W)
    _fill(ones_v, 1.0, DEG_W)
    plsc.subcore_barrier()

    def chunk(ci, _):
        pltpu.sync_copy(idx_hbm.at[wid, ci, 1], dst0)
        pltpu.sync_copy(ones_v, acc_sh.at[dst0], add=True)
        return 0

    lax.fori_loop(0, NCHUNKS, chunk, 0)
    plsc.subcore_barrier()
    _copy_out(acc_sh, out_hbm, c, s)


ROWS_BLK = 2000


def _invdeg_body(dp_ref, o_ref):
    deg = dp_ref[0, :, :1] + dp_ref[1, :, :1]
    o_ref[...] = jnp.broadcast_to(1.0 / jnp.maximum(deg, 1.0), (ROWS_BLK, D))


def _tc_invdeg(deg_p):
    return pl.pallas_call(
        _invdeg_body,
        grid=(N // ROWS_BLK,),
        in_specs=[pl.BlockSpec((NC, ROWS_BLK, DEG_W), lambda i: (0, i, 0))],
        out_specs=pl.BlockSpec((ROWS_BLK, D), lambda i: (i, 0)),
        out_shape=jax.ShapeDtypeStruct((N, D), jnp.float32),
    )(deg_p)


def _layer_body(relu, p_ref, h_ref, inv_ref, wl_ref, wr_ref, b_ref, o_ref):
    agg = (p_ref[0] + p_ref[1]) * inv_ref[...]
    dn = (((1,), (1,)), ((), ()))
    acc = lax.dot_general(agg, wl_ref[...], dn, preferred_element_type=jnp.float32)
    acc = acc + lax.dot_general(h_ref[...], wr_ref[...], dn,
                                preferred_element_type=jnp.float32)
    acc = acc + b_ref[...]
    o_ref[...] = jnp.maximum(acc, 0.0) if relu else acc


def _tc_layer(p, h, invd, wl, wr, bb, relu):
    return pl.pallas_call(
        functools.partial(_layer_body, relu),
        grid=(N // ROWS_BLK,),
        in_specs=[
            pl.BlockSpec((NC, ROWS_BLK, D), lambda i: (0, i, 0)),
            pl.BlockSpec((ROWS_BLK, D), lambda i: (i, 0)),
            pl.BlockSpec((ROWS_BLK, D), lambda i: (i, 0)),
            pl.BlockSpec((D, D), lambda i: (0, 0)),
            pl.BlockSpec((D, D), lambda i: (0, 0)),
            pl.BlockSpec((1, D), lambda i: (0, 0)),
        ],
        out_specs=pl.BlockSpec((ROWS_BLK, D), lambda i: (i, 0)),
        out_shape=jax.ShapeDtypeStruct((N, D), jnp.float32),
    )(p, h, invd, wl, wr, bb)


def kernel(x, edge_index, Wl, Wr, b):
    src = edge_index[0].astype(jnp.int32)
    dst = edge_index[1].astype(jnp.int32)
    pad = E_PAD - E
    src_p = jnp.concatenate([src, jnp.zeros((pad,), jnp.int32)])
    dst_p = jnp.concatenate([dst, jnp.full((pad,), N, jnp.int32)])
    idx = jnp.stack([src_p.reshape(NW, NCHUNKS, CHUNK),
                     dst_p.reshape(NW, NCHUNKS, CHUNK)], axis=2)

    deg_p = _sc_deg(idx)
    invd = _tc_invdeg(deg_p)

    h = x
    for i in range(L):
        p = _sc_agg(jnp.concatenate([h, h], axis=1), idx)
        h = _tc_layer(p, h, invd, Wl[i], Wr[i], b[i][None, :], relu=(i < L - 1))
    return h


# speedup vs baseline: 1.0692x; 1.0692x over previous
import functools

import jax
import jax.numpy as jnp
from jax import lax
from jax.experimental import pallas as pl
from jax.experimental.pallas import tpu as pltpu
from jax.experimental.pallas import tpu_sc as plsc

N = 10000
E = 320000
D = 128
L = 10

NC = 2
NS = 16
NW = NC * NS
CHUNK = 80
NCHUNKS = 128
TOTC = NW * NCHUNKS
EPT = CHUNK * NCHUNKS
E_PAD = EPT * NW
N_PAD = 10240
SLAB = N_PAD // NS
LAST = N - (NS - 1) * SLAB
SPAN = 64
DEG_W = D

_MESH = plsc.VectorSubcoreMesh(
    core_axis_name="c", subcore_axis_name="s", num_cores=NC, num_subcores=NS
)


def _fill(buf, val, width=D):
    vec = jnp.full((16,), val, jnp.float32)

    def body(r, _):
        for k in range(width // 16):
            buf[r, pl.ds(k * 16, 16)] = vec
        return 0

    lax.fori_loop(0, CHUNK, body, 0)


def _zero_slab(zbuf, acc_sh, s, width=D):
    _fill(zbuf, 0.0, width)
    slab = pl.multiple_of(s * SLAB, CHUNK)
    for k in range(SLAB // CHUNK):
        pltpu.sync_copy(zbuf, acc_sh.at[pl.ds(slab + k * CHUNK, CHUNK)])


def _copy_out(acc_sh, out_hbm, c, s):
    start = pl.multiple_of(s * SLAB, CHUNK)

    @pl.when(s < NS - 1)
    def _():
        pltpu.sync_copy(acc_sh.at[pl.ds(start, SLAB)],
                        out_hbm.at[c, pl.ds(start, SLAB)])

    @pl.when(s == NS - 1)
    def _():
        pltpu.sync_copy(acc_sh.at[pl.ds(start, LAST)],
                        out_hbm.at[c, pl.ds(start, LAST)])


@functools.partial(
    pl.kernel,
    out_type=jax.ShapeDtypeStruct((NC, N, D), jnp.float32),
    mesh=_MESH,
    scratch_types=[
        [pltpu.VMEM((SPAN, D), jnp.float32)] * 2,
        [pltpu.VMEM((CHUNK, D), jnp.float32)] * 2,
        [pltpu.VMEM((CHUNK + 16,), jnp.int32)] * 2,
        [pltpu.VMEM((CHUNK,), jnp.int32)] * 2,
        pltpu.VMEM((NCHUNKS + 16,), jnp.int32),
        pltpu.VMEM((32,), jnp.int32),
        pltpu.VMEM((CHUNK,), jnp.int32),
        pltpu.VMEM_SHARED((N_PAD, D), jnp.float32),
        [pltpu.SemaphoreType.DMA] * 2,
        [pltpu.SemaphoreType.DMA] * 2,
        [pltpu.SemaphoreType.DMA] * 2,
        [pltpu.SemaphoreType.DMA] * 2,
        pltpu.SemaphoreType.DMA,
    ],
)
def _sc_agg(h_hbm, gidx_hbm, splat_hbm, base_hbm, cnt_hbm, out_hbm,
            span, msg, spl, dstb, base_v, cnt_v, fbsrc, acc_sh,
            psems, isems, dsems, ssems, fsem):
    c = lax.axis_index("c")
    s = lax.axis_index("s")
    wid = s * NC + c

    pltpu.sync_copy(base_hbm.at[wid], base_v.at[pl.ds(0, NCHUNKS)])
    pltpu.sync_copy(cnt_hbm.at[pl.ds((wid // 16) * 16, 16)],
                    cnt_v.at[pl.ds(0, 16)])
    fast_cnt = cnt_v[pl.ds(wid % 16, 16)][0]

    def chunk_base(k):
        return pl.multiple_of(base_v[pl.ds(k, 16)][0], 8)

    _zero_slab(msg[0], acc_sh, s)
    b0 = chunk_base(0)
    pltpu.async_copy(h_hbm.at[pl.ds(b0, SPAN)], span[0], psems[0])
    pltpu.async_copy(splat_hbm.at[wid, 0], spl[0], isems[0])
    pltpu.async_copy(gidx_hbm.at[wid, 0, 1], dstb[0], dsems[0])
    plsc.subcore_barrier()

    cols = [lax.iota(jnp.int32, 16) + 16 * kk for kk in range(D // 16)]

    def fast_pair(g, _):
        for b in range(2):
            k = 2 * g + b
            nb = 1 - b
            if b == 0:
                @pl.when(g >= 1)
                def _():
                    pltpu.make_async_copy(msg[nb], acc_sh.at[dstb[nb]],
                                          ssems[nb]).wait()
            else:
                pltpu.make_async_copy(msg[nb], acc_sh.at[dstb[nb]],
                                      ssems[nb]).wait()
            kn = jnp.minimum(k + 1, NCHUNKS - 1)
            bn = chunk_base(kn)
            pltpu.async_copy(h_hbm.at[pl.ds(bn, SPAN)], span[nb], psems[nb])
            pltpu.async_copy(splat_hbm.at[wid, kn], spl[nb], isems[nb])
            pltpu.async_copy(gidx_hbm.at[wid, kn, 1], dstb[nb], dsems[nb])
            pltpu.make_async_copy(h_hbm.at[pl.ds(bn, SPAN)], span[b],
                                  psems[b]).wait()
            pltpu.make_async_copy(splat_hbm.at[wid, 0], spl[b],
                                  isems[b]).wait()
            pltpu.make_async_copy(gidx_hbm.at[wid, 0, 1], dstb[b],
                                  dsems[b]).wait()

            def edge(e, _):
                r = spl[b][pl.ds(e, 16)][0]
                for kk in range(D // 16):
                    msg[b][e, pl.ds(16 * kk, 16)] = span[b][r, pl.ds(16 * kk, 16)]
                return 0

            lax.fori_loop(0, CHUNK, edge, 0)
            pltpu.async_copy(msg[b], acc_sh.at[dstb[b]], ssems[b], add=True)
        return 0

    lax.fori_loop(0, lax.div(fast_cnt, 2), fast_pair, 0)

    @pl.when(fast_cnt > 0)
    def _():
        pltpu.make_async_copy(msg[1], acc_sh.at[dstb[1]], ssems[1]).wait()
        pltpu.make_async_copy(gidx_hbm.at[wid, 0, 1], dstb[0],
                              dsems[0]).wait()
    pltpu.make_async_copy(h_hbm.at[pl.ds(b0, SPAN)], span[0], psems[0]).wait()
    pltpu.make_async_copy(splat_hbm.at[wid, 0], spl[0], isems[0]).wait()

    @pl.when(fast_cnt == 0)
    def _():
        pltpu.make_async_copy(gidx_hbm.at[wid, 0, 1], dstb[0],
                              dsems[0]).wait()

    def fb_chunk(k, _):
        pltpu.sync_copy(gidx_hbm.at[wid, k, 0], fbsrc)
        pltpu.sync_copy(gidx_hbm.at[wid, k, 1], dstb[0])
        pltpu.async_copy(h_hbm.at[fbsrc], msg[0], fsem).wait()
        pltpu.sync_copy(msg[0], acc_sh.at[dstb[0]], add=True)
        return 0

    lax.fori_loop(fast_cnt, NCHUNKS, fb_chunk, 0)

    plsc.subcore_barrier()
    _copy_out(acc_sh, out_hbm, c, s)


@functools.partial(
    pl.kernel,
    out_type=jax.ShapeDtypeStruct((NC, N, DEG_W), jnp.float32),
    mesh=_MESH,
    scratch_types=[
        pltpu.VMEM((CHUNK, DEG_W), jnp.float32),
        pltpu.VMEM((CHUNK,), jnp.int32),
        pltpu.VMEM_SHARED((N_PAD, DEG_W), jnp.float32),
        pltpu.SemaphoreType.DMA,
    ],
)
def _sc_deg(idx_hbm, out_hbm, ones_v, dst0, acc_sh, sem):
    c = lax.axis_index("c")
    s = lax.axis_index("s")
    wid = s * NC + c

    _zero_slab(ones_v, acc_sh, s, DEG_W)
    _fill(ones_v, 1.0, DEG_W)
    plsc.subcore_barrier()

    def chunk(ci, _):
        pltpu.sync_copy(idx_hbm.at[wid, ci, 1], dst0)
        pltpu.sync_copy(ones_v, acc_sh.at[dst0], add=True)
        return 0

    lax.fori_loop(0, NCHUNKS, chunk, 0)
    plsc.subcore_barrier()
    _copy_out(acc_sh, out_hbm, c, s)


ROWS_BLK = 2000


def _invdeg_body(dp_ref, o_ref):
    deg = dp_ref[0, :, :1] + dp_ref[1, :, :1]
    o_ref[...] = jnp.broadcast_to(1.0 / jnp.maximum(deg, 1.0), (ROWS_BLK, D))


def _tc_invdeg(deg_p):
    return pl.pallas_call(
        _invdeg_body,
        grid=(N // ROWS_BLK,),
        in_specs=[pl.BlockSpec((NC, ROWS_BLK, DEG_W), lambda i: (0, i, 0))],
        out_specs=pl.BlockSpec((ROWS_BLK, D), lambda i: (i, 0)),
        out_shape=jax.ShapeDtypeStruct((N, D), jnp.float32),
    )(deg_p)


def _layer_body(relu, p_ref, h_ref, inv_ref, wl_ref, wr_ref, b_ref, o_ref):
    agg = (p_ref[0] + p_ref[1]) * inv_ref[...]
    dn = (((1,), (1,)), ((), ()))
    acc = lax.dot_general(agg, wl_ref[...], dn, preferred_element_type=jnp.float32)
    acc = acc + lax.dot_general(h_ref[...], wr_ref[...], dn,
                                preferred_element_type=jnp.float32)
    acc = acc + b_ref[...]
    o_ref[...] = jnp.maximum(acc, 0.0) if relu else acc


def _tc_layer(p, hp, invd, wl, wr, bb, relu):
    return pl.pallas_call(
        functools.partial(_layer_body, relu),
        grid=(N // ROWS_BLK,),
        in_specs=[
            pl.BlockSpec((NC, ROWS_BLK, D), lambda i: (0, i, 0)),
            pl.BlockSpec((ROWS_BLK, D), lambda i: (i, 0)),
            pl.BlockSpec((ROWS_BLK, D), lambda i: (i, 0)),
            pl.BlockSpec((D, D), lambda i: (0, 0)),
            pl.BlockSpec((D, D), lambda i: (0, 0)),
            pl.BlockSpec((1, D), lambda i: (0, 0)),
        ],
        out_specs=pl.BlockSpec((ROWS_BLK, D), lambda i: (i, 0)),
        out_shape=jax.ShapeDtypeStruct((N_PAD, D), jnp.float32),
    )(p, hp, invd, wl, wr, bb)


def kernel(x, edge_index, Wl, Wr, b):
    src = edge_index[0].astype(jnp.int32)
    dst = edge_index[1].astype(jnp.int32)
    pad = E_PAD - E
    src_p = jnp.concatenate([src, jnp.full((pad,), N, jnp.int32)])
    dst_p = jnp.concatenate([dst, jnp.full((pad,), N, jnp.int32)])

    order = jnp.argsort(src_p)
    ss = src_p[order].reshape(TOTC, CHUNK)
    dd = dst_p[order].reshape(TOTC, CHUNK)
    first = ss[:, 0]
    last = ss[:, -1]
    base = jnp.minimum(first, N_PAD - SPAN).astype(jnp.int32)
    base = base - base % 8
    ok = (last - base) < SPAN
    src_local = jnp.clip(ss - base[:, None], 0, SPAN - 1).astype(jnp.int32)

    okt = ok.reshape(NW, NCHUNKS)
    perm = jnp.argsort(~okt, axis=1)
    fast_cnt = okt.sum(axis=1).astype(jnp.int32)
    fast_cnt = fast_cnt - fast_cnt % 2

    gidx = jnp.stack([ss, dd], axis=1).reshape(NW, NCHUNKS, 2, CHUNK)
    gidx = jnp.take_along_axis(gidx, perm[:, :, None, None], axis=1)
    splat = jnp.pad(src_local, ((0, 0), (0, 16))).reshape(NW, NCHUNKS, CHUNK + 16)
    splat = jnp.take_along_axis(splat, perm[:, :, None], axis=1)
    base_t = jnp.take_along_axis(base.reshape(NW, NCHUNKS), perm, axis=1)

    deg_p = _sc_deg(gidx)
    invd = _tc_invdeg(deg_p)

    hp = jnp.concatenate([x, jnp.zeros((N_PAD - N, D), jnp.float32)])
    for i in range(L):
        p = _sc_agg(hp, gidx, splat, base_t, fast_cnt)
        hp = _tc_layer(p, hp, invd, Wl[i], Wr[i], b[i][None, :],
                       relu=(i < L - 1))
    return hp[:N]


# speedup vs baseline: 1.8250x; 1.7069x over previous
import functools

import jax
import jax.numpy as jnp
from jax import lax
from jax.experimental import pallas as pl
from jax.experimental.pallas import tpu as pltpu
from jax.experimental.pallas import tpu_sc as plsc

N = 10000
E = 320000
D = 128
L = 10

NC = 2
NS = 16
NW = NC * NS
CHUNK = 80
NCHUNKS = 128
TOTC = NW * NCHUNKS
EPT = CHUNK * NCHUNKS
E_PAD = EPT * NW
N_PAD = 10240
SLAB = N_PAD // NS
LAST = N - (NS - 1) * SLAB
SPAN = 64
DEG_W = D

_MESH = plsc.VectorSubcoreMesh(
    core_axis_name="c", subcore_axis_name="s", num_cores=NC, num_subcores=NS
)


def _fill(buf, val, width=D):
    vec = jnp.full((16,), val, jnp.float32)

    def body(r, _):
        for k in range(width // 16):
            buf[r, pl.ds(k * 16, 16)] = vec
        return 0

    lax.fori_loop(0, CHUNK, body, 0)


def _zero_slab(zbuf, acc_sh, s, width=D):
    _fill(zbuf, 0.0, width)
    slab = pl.multiple_of(s * SLAB, CHUNK)
    for k in range(SLAB // CHUNK):
        pltpu.sync_copy(zbuf, acc_sh.at[pl.ds(slab + k * CHUNK, CHUNK)])


def _copy_out(acc_sh, out_hbm, c, s):
    start = pl.multiple_of(s * SLAB, CHUNK)

    @pl.when(s < NS - 1)
    def _():
        pltpu.sync_copy(acc_sh.at[pl.ds(start, SLAB)],
                        out_hbm.at[c, pl.ds(start, SLAB)])

    @pl.when(s == NS - 1)
    def _():
        pltpu.sync_copy(acc_sh.at[pl.ds(start, LAST)],
                        out_hbm.at[c, pl.ds(start, LAST)])


@functools.partial(
    pl.kernel,
    out_type=jax.ShapeDtypeStruct((NC, N, D), jnp.float32),
    mesh=_MESH,
    scratch_types=[
        [pltpu.VMEM((SPAN, D), jnp.float32)] * 2,
        [pltpu.VMEM((CHUNK, D), jnp.float32)] * 2,
        [pltpu.VMEM((2, 96), jnp.int32)] * 2,
        [pltpu.VMEM((CHUNK,), jnp.int32)] * 2,
        pltpu.VMEM((NCHUNKS + 16,), jnp.int32),
        pltpu.VMEM((NCHUNKS + 16,), jnp.int32),
        pltpu.VMEM((32,), jnp.int32),
        pltpu.VMEM((CHUNK,), jnp.int32),
        pltpu.VMEM_SHARED((N_PAD, D), jnp.float32),
        [pltpu.SemaphoreType.DMA] * 2,
        [pltpu.SemaphoreType.DMA] * 2,
        [pltpu.SemaphoreType.DMA] * 2,
        [pltpu.SemaphoreType.DMA] * 2,
        pltpu.SemaphoreType.DMA,
    ],
)
def _sc_agg(h_hbm, gidx_hbm, rmeta_hbm, base_hbm, nruns_hbm, cnt_hbm, out_hbm,
            span, msg, rm, dstb, base_v, nruns_v, cnt_v, fbsrc, acc_sh,
            psems, isems, dsems, ssems, fsem):
    c = lax.axis_index("c")
    s = lax.axis_index("s")
    wid = s * NC + c

    pltpu.sync_copy(base_hbm.at[wid], base_v.at[pl.ds(0, NCHUNKS)])
    pltpu.sync_copy(nruns_hbm.at[wid], nruns_v.at[pl.ds(0, NCHUNKS)])
    pltpu.sync_copy(cnt_hbm.at[pl.ds((wid // 16) * 16, 16)],
                    cnt_v.at[pl.ds(0, 16)])
    fast_cnt = cnt_v[pl.ds(wid % 16, 16)][0]

    def chunk_base(k):
        return pl.multiple_of(base_v[pl.ds(k, 16)][0], 8)

    _zero_slab(msg[0], acc_sh, s)
    b0 = chunk_base(0)
    pltpu.async_copy(h_hbm.at[pl.ds(b0, SPAN)], span[0], psems[0])
    pltpu.async_copy(rmeta_hbm.at[wid, 0], rm[0], isems[0])
    pltpu.async_copy(gidx_hbm.at[wid, 0, 1], dstb[0], dsems[0])
    plsc.subcore_barrier()

    cols = [lax.iota(jnp.int32, 16) + 16 * kk for kk in range(D // 16)]

    def fast_pair(g, _):
        for b in range(2):
            k = 2 * g + b
            nb = 1 - b
            if b == 0:
                @pl.when(g >= 1)
                def _():
                    pltpu.make_async_copy(msg[nb], acc_sh.at[dstb[nb]],
                                          ssems[nb]).wait()
            else:
                pltpu.make_async_copy(msg[nb], acc_sh.at[dstb[nb]],
                                      ssems[nb]).wait()
            kn = jnp.minimum(k + 1, NCHUNKS - 1)
            bn = chunk_base(kn)
            pltpu.async_copy(h_hbm.at[pl.ds(bn, SPAN)], span[nb], psems[nb])
            pltpu.async_copy(rmeta_hbm.at[wid, kn], rm[nb], isems[nb])
            pltpu.async_copy(gidx_hbm.at[wid, kn, 1], dstb[nb], dsems[nb])
            pltpu.make_async_copy(h_hbm.at[pl.ds(bn, SPAN)], span[b],
                                  psems[b]).wait()
            pltpu.make_async_copy(rmeta_hbm.at[wid, 0], rm[b],
                                  isems[b]).wait()
            pltpu.make_async_copy(gidx_hbm.at[wid, 0, 1], dstb[b],
                                  dsems[b]).wait()

            nr = nruns_v[pl.ds(k, 16)][0]

            def run(j, pos):
                r = rm[b][0, pl.ds(j, 16)][0]
                ln = rm[b][1, pl.ds(j, 16)][0]
                vs = [span[b][r, pl.ds(16 * kk, 16)] for kk in range(D // 16)]

                def put(i, _):
                    for kk in range(D // 16):
                        msg[b][pos + i, pl.ds(16 * kk, 16)] = vs[kk]
                    return 0

                lax.fori_loop(0, ln, put, 0)
                return pos + ln

            lax.fori_loop(0, nr, run, 0)
            pltpu.async_copy(msg[b], acc_sh.at[dstb[b]], ssems[b], add=True)
        return 0

    lax.fori_loop(0, lax.div(fast_cnt, 2), fast_pair, 0)

    @pl.when(fast_cnt > 0)
    def _():
        pltpu.make_async_copy(msg[1], acc_sh.at[dstb[1]], ssems[1]).wait()
        pltpu.make_async_copy(gidx_hbm.at[wid, 0, 1], dstb[0],
                              dsems[0]).wait()
    pltpu.make_async_copy(h_hbm.at[pl.ds(b0, SPAN)], span[0], psems[0]).wait()
    pltpu.make_async_copy(rmeta_hbm.at[wid, 0], rm[0], isems[0]).wait()

    @pl.when(fast_cnt == 0)
    def _():
        pltpu.make_async_copy(gidx_hbm.at[wid, 0, 1], dstb[0],
                              dsems[0]).wait()

    def fb_chunk(k, _):
        pltpu.sync_copy(gidx_hbm.at[wid, k, 0], fbsrc)
        pltpu.sync_copy(gidx_hbm.at[wid, k, 1], dstb[0])
        pltpu.async_copy(h_hbm.at[fbsrc], msg[0], fsem).wait()
        pltpu.sync_copy(msg[0], acc_sh.at[dstb[0]], add=True)
        return 0

    lax.fori_loop(fast_cnt, NCHUNKS, fb_chunk, 0)

    plsc.subcore_barrier()
    _copy_out(acc_sh, out_hbm, c, s)


@functools.partial(
    pl.kernel,
    out_type=jax.ShapeDtypeStruct((NC, N, DEG_W), jnp.float32),
    mesh=_MESH,
    scratch_types=[
        pltpu.VMEM((CHUNK, DEG_W), jnp.float32),
        pltpu.VMEM((CHUNK,), jnp.int32),
        pltpu.VMEM_SHARED((N_PAD, DEG_W), jnp.float32),
        pltpu.SemaphoreType.DMA,
    ],
)
def _sc_deg(idx_hbm, out_hbm, ones_v, dst0, acc_sh, sem):
    c = lax.axis_index("c")
    s = lax.axis_index("s")
    wid = s * NC + c

    _zero_slab(ones_v, acc_sh, s, DEG_W)
    _fill(ones_v, 1.0, DEG_W)
    plsc.subcore_barrier()

    def chunk(ci, _):
        pltpu.sync_copy(idx_hbm.at[wid, ci, 1], dst0)
        pltpu.sync_copy(ones_v, acc_sh.at[dst0], add=True)
        return 0

    lax.fori_loop(0, NCHUNKS, chunk, 0)
    plsc.subcore_barrier()
    _copy_out(acc_sh, out_hbm, c, s)


ROWS_BLK = 2000


def _invdeg_body(dp_ref, o_ref):
    deg = dp_ref[0, :, :1] + dp_ref[1, :, :1]
    o_ref[...] = jnp.broadcast_to(1.0 / jnp.maximum(deg, 1.0), (ROWS_BLK, D))


def _tc_invdeg(deg_p):
    return pl.pallas_call(
        _invdeg_body,
        grid=(N // ROWS_BLK,),
        in_specs=[pl.BlockSpec((NC, ROWS_BLK, DEG_W), lambda i: (0, i, 0))],
        out_specs=pl.BlockSpec((ROWS_BLK, D), lambda i: (i, 0)),
        out_shape=jax.ShapeDtypeStruct((N, D), jnp.float32),
    )(deg_p)


def _layer_body(relu, p_ref, h_ref, inv_ref, wl_ref, wr_ref, b_ref, o_ref):
    agg = (p_ref[0] + p_ref[1]) * inv_ref[...]
    dn = (((1,), (1,)), ((), ()))
    acc = lax.dot_general(agg, wl_ref[...], dn, preferred_element_type=jnp.float32)
    acc = acc + lax.dot_general(h_ref[...], wr_ref[...], dn,
                                preferred_element_type=jnp.float32)
    acc = acc + b_ref[...]
    o_ref[...] = jnp.maximum(acc, 0.0) if relu else acc


def _tc_layer(p, hp, invd, wl, wr, bb, relu):
    return pl.pallas_call(
        functools.partial(_layer_body, relu),
        grid=(N // ROWS_BLK,),
        in_specs=[
            pl.BlockSpec((NC, ROWS_BLK, D), lambda i: (0, i, 0)),
            pl.BlockSpec((ROWS_BLK, D), lambda i: (i, 0)),
            pl.BlockSpec((ROWS_BLK, D), lambda i: (i, 0)),
            pl.BlockSpec((D, D), lambda i: (0, 0)),
            pl.BlockSpec((D, D), lambda i: (0, 0)),
            pl.BlockSpec((1, D), lambda i: (0, 0)),
        ],
        out_specs=pl.BlockSpec((ROWS_BLK, D), lambda i: (i, 0)),
        out_shape=jax.ShapeDtypeStruct((N_PAD, D), jnp.float32),
    )(p, hp, invd, wl, wr, bb)


def kernel(x, edge_index, Wl, Wr, b):
    src = edge_index[0].astype(jnp.int32)
    dst = edge_index[1].astype(jnp.int32)
    pad = E_PAD - E
    src_p = jnp.concatenate([src, jnp.full((pad,), N, jnp.int32)])
    dst_p = jnp.concatenate([dst, jnp.full((pad,), N, jnp.int32)])

    order = jnp.argsort(src_p)
    ss = src_p[order].reshape(TOTC, CHUNK)
    dd = dst_p[order].reshape(TOTC, CHUNK)
    first = ss[:, 0]
    last = ss[:, -1]
    base = jnp.minimum(first, N_PAD - SPAN).astype(jnp.int32)
    base = base - base % 8
    ok = (last - base) < SPAN
    src_local = jnp.clip(ss - base[:, None], 0, SPAN - 1).astype(jnp.int32)

    okt = ok.reshape(NW, NCHUNKS)
    perm = jnp.argsort(~okt, axis=1)
    fast_cnt = okt.sum(axis=1).astype(jnp.int32)
    fast_cnt = fast_cnt - fast_cnt % 2

    gidx = jnp.stack([ss, dd], axis=1).reshape(NW, NCHUNKS, 2, CHUNK)
    gidx = jnp.take_along_axis(gidx, perm[:, :, None, None], axis=1)
    is_start = jnp.concatenate(
        [jnp.ones((TOTC, 1), bool), src_local[:, 1:] != src_local[:, :-1]],
        axis=1)
    rid = jnp.cumsum(is_start, axis=1) - 1
    nruns = (rid[:, -1] + 1).astype(jnp.int32)
    rows_ix = jnp.arange(TOTC)[:, None]
    runrow = jnp.zeros((TOTC, 96), jnp.int32).at[rows_ix, rid].set(src_local)
    runlen = jnp.zeros((TOTC, 96), jnp.int32).at[rows_ix, rid].add(1)
    rmeta = jnp.stack([runrow, runlen], axis=1).reshape(NW, NCHUNKS, 2, 96)
    rmeta = jnp.take_along_axis(rmeta, perm[:, :, None, None], axis=1)
    nruns_t = jnp.take_along_axis(nruns.reshape(NW, NCHUNKS), perm, axis=1)
    base_t = jnp.take_along_axis(base.reshape(NW, NCHUNKS), perm, axis=1)

    deg_p = _sc_deg(gidx)
    invd = _tc_invdeg(deg_p)

    hp = jnp.concatenate([x, jnp.zeros((N_PAD - N, D), jnp.float32)])
    for i in range(L):
        p = _sc_agg(hp, gidx, rmeta, base_t, nruns_t, fast_cnt)
        hp = _tc_layer(p, hp, invd, Wl[i], Wr[i], b[i][None, :],
                       relu=(i < L - 1))
    return hp[:N]


# speedup vs baseline: 1.8577x; 1.0179x over previous
import functools

import jax
import jax.numpy as jnp
from jax import lax
from jax.experimental import pallas as pl
from jax.experimental.pallas import tpu as pltpu
from jax.experimental.pallas import tpu_sc as plsc

N = 10000
E = 320000
D = 128
L = 10

NC = 2
NS = 16
NW = NC * NS
CHUNK = 80
NCHUNKS = 128
TOTC = NW * NCHUNKS
EPT = CHUNK * NCHUNKS
E_PAD = EPT * NW
N_PAD = 10240
SLAB = N_PAD // NS
LAST = N - (NS - 1) * SLAB
SPAN = 64
DEG_W = D

_MESH = plsc.VectorSubcoreMesh(
    core_axis_name="c", subcore_axis_name="s", num_cores=NC, num_subcores=NS
)


def _fill(buf, val, width=D):
    vec = jnp.full((16,), val, jnp.float32)

    def body(r, _):
        for k in range(width // 16):
            buf[r, pl.ds(k * 16, 16)] = vec
        return 0

    lax.fori_loop(0, CHUNK, body, 0)


def _zero_slab(zbuf, acc_sh, s, width=D):
    _fill(zbuf, 0.0, width)
    slab = pl.multiple_of(s * SLAB, CHUNK)
    for k in range(SLAB // CHUNK):
        pltpu.sync_copy(zbuf, acc_sh.at[pl.ds(slab + k * CHUNK, CHUNK)])


def _copy_out(acc_sh, out_hbm, c, s):
    start = pl.multiple_of(s * SLAB, CHUNK)

    @pl.when(s < NS - 1)
    def _():
        pltpu.sync_copy(acc_sh.at[pl.ds(start, SLAB)],
                        out_hbm.at[c, pl.ds(start, SLAB)])

    @pl.when(s == NS - 1)
    def _():
        pltpu.sync_copy(acc_sh.at[pl.ds(start, LAST)],
                        out_hbm.at[c, pl.ds(start, LAST)])


@functools.partial(
    pl.kernel,
    out_type=jax.ShapeDtypeStruct((NC, N, D), jnp.float32),
    mesh=_MESH,
    scratch_types=[
        [pltpu.VMEM((SPAN, D), jnp.float32)] * 2,
        [pltpu.VMEM((CHUNK, D), jnp.float32)] * 2,
        [pltpu.VMEM((2, 96), jnp.int32)] * 2,
        [pltpu.VMEM((CHUNK,), jnp.int32)] * 2,
        pltpu.VMEM((NCHUNKS + 16,), jnp.int32),
        pltpu.VMEM((NCHUNKS + 16,), jnp.int32),
        pltpu.VMEM((32,), jnp.int32),
        pltpu.VMEM((CHUNK,), jnp.int32),
        pltpu.VMEM_SHARED((N_PAD, D), jnp.float32),
        [pltpu.SemaphoreType.DMA] * 2,
        [pltpu.SemaphoreType.DMA] * 2,
        [pltpu.SemaphoreType.DMA] * 2,
        [pltpu.SemaphoreType.DMA] * 2,
        pltpu.SemaphoreType.DMA,
    ],
)
def _sc_agg(h_hbm, gidx_hbm, rmeta_hbm, base_hbm, nruns_hbm, cnt_hbm, out_hbm,
            span, msg, rm, dstb, base_v, nruns_v, cnt_v, fbsrc, acc_sh,
            psems, isems, dsems, ssems, fsem):
    c = lax.axis_index("c")
    s = lax.axis_index("s")
    wid = s * NC + c

    pltpu.sync_copy(base_hbm.at[wid], base_v.at[pl.ds(0, NCHUNKS)])
    pltpu.sync_copy(nruns_hbm.at[wid], nruns_v.at[pl.ds(0, NCHUNKS)])
    pltpu.sync_copy(cnt_hbm.at[pl.ds((wid // 16) * 16, 16)],
                    cnt_v.at[pl.ds(0, 16)])
    fast_cnt = cnt_v[pl.ds(wid % 16, 16)][0]

    def chunk_base(k):
        return pl.multiple_of(base_v[pl.ds(k, 16)][0], 8)

    _zero_slab(msg[0], acc_sh, s)
    b0 = chunk_base(0)
    pltpu.async_copy(h_hbm.at[pl.ds(b0, SPAN)], span[0], psems[0])
    pltpu.async_copy(rmeta_hbm.at[wid, 0], rm[0], isems[0])
    pltpu.async_copy(gidx_hbm.at[wid, 0, 1], dstb[0], dsems[0])
    plsc.subcore_barrier()

    cols = [lax.iota(jnp.int32, 16) + 16 * kk for kk in range(D // 16)]

    def fast_pair(g, _):
        for b in range(2):
            k = 2 * g + b
            nb = 1 - b
            if b == 0:
                @pl.when(g >= 1)
                def _():
                    pltpu.make_async_copy(msg[nb], acc_sh.at[dstb[nb]],
                                          ssems[nb]).wait()
            else:
                pltpu.make_async_copy(msg[nb], acc_sh.at[dstb[nb]],
                                      ssems[nb]).wait()
            kn = jnp.minimum(k + 1, NCHUNKS - 1)
            bn = chunk_base(kn)
            pltpu.async_copy(h_hbm.at[pl.ds(bn, SPAN)], span[nb], psems[nb])
            pltpu.async_copy(rmeta_hbm.at[wid, kn], rm[nb], isems[nb])
            pltpu.async_copy(gidx_hbm.at[wid, kn, 1], dstb[nb], dsems[nb])
            pltpu.make_async_copy(h_hbm.at[pl.ds(bn, SPAN)], span[b],
                                  psems[b]).wait()
            pltpu.make_async_copy(rmeta_hbm.at[wid, 0], rm[b],
                                  isems[b]).wait()
            pltpu.make_async_copy(gidx_hbm.at[wid, 0, 1], dstb[b],
                                  dsems[b]).wait()

            nr = nruns_v[pl.ds(k, 16)][0]

            def run(j, pos):
                r = rm[b][0, pl.ds(j, 16)][0]
                ln = rm[b][1, pl.ds(j, 16)][0]
                vs = [span[b][r, pl.ds(16 * kk, 16)] for kk in range(D // 16)]

                def put(i, _):
                    for kk in range(D // 16):
                        msg[b][pos + i, pl.ds(16 * kk, 16)] = vs[kk]
                    return 0

                lax.fori_loop(0, ln, put, 0)
                return pos + ln

            lax.fori_loop(0, nr, run, 0)
            pltpu.async_copy(msg[b], acc_sh.at[dstb[b]], ssems[b], add=True)
        return 0

    lax.fori_loop(0, lax.div(fast_cnt, 2), fast_pair, 0)

    @pl.when(fast_cnt > 0)
    def _():
        pltpu.make_async_copy(msg[1], acc_sh.at[dstb[1]], ssems[1]).wait()
        pltpu.make_async_copy(gidx_hbm.at[wid, 0, 1], dstb[0],
                              dsems[0]).wait()
    pltpu.make_async_copy(h_hbm.at[pl.ds(b0, SPAN)], span[0], psems[0]).wait()
    pltpu.make_async_copy(rmeta_hbm.at[wid, 0], rm[0], isems[0]).wait()

    @pl.when(fast_cnt == 0)
    def _():
        pltpu.make_async_copy(gidx_hbm.at[wid, 0, 1], dstb[0],
                              dsems[0]).wait()

    def fb_chunk(k, _):
        pltpu.sync_copy(gidx_hbm.at[wid, k, 0], fbsrc)
        pltpu.sync_copy(gidx_hbm.at[wid, k, 1], dstb[0])
        pltpu.async_copy(h_hbm.at[fbsrc], msg[0], fsem).wait()
        pltpu.sync_copy(msg[0], acc_sh.at[dstb[0]], add=True)
        return 0

    lax.fori_loop(fast_cnt, NCHUNKS, fb_chunk, 0)

    plsc.subcore_barrier()
    _copy_out(acc_sh, out_hbm, c, s)


@functools.partial(
    pl.kernel,
    out_type=jax.ShapeDtypeStruct((NC, N, DEG_W), jnp.float32),
    mesh=_MESH,
    scratch_types=[
        pltpu.VMEM((CHUNK, DEG_W), jnp.float32),
        [pltpu.VMEM((CHUNK,), jnp.int32)] * 2,
        pltpu.VMEM_SHARED((N_PAD, DEG_W), jnp.float32),
        [pltpu.SemaphoreType.DMA] * 2,
        [pltpu.SemaphoreType.DMA] * 2,
    ],
)
def _sc_deg(idx_hbm, out_hbm, ones_v, dstb, acc_sh, dsems, ssems):
    c = lax.axis_index("c")
    s = lax.axis_index("s")
    wid = s * NC + c

    pltpu.async_copy(idx_hbm.at[wid, 0, 1], dstb[0], dsems[0])
    _zero_slab(ones_v, acc_sh, s, DEG_W)
    _fill(ones_v, 1.0, DEG_W)
    plsc.subcore_barrier()

    def pair(g, _):
        for b in range(2):
            k = 2 * g + b
            nb = 1 - b
            if b == 0:
                @pl.when(g >= 1)
                def _():
                    pltpu.make_async_copy(ones_v, acc_sh.at[dstb[nb]],
                                          ssems[nb]).wait()
            else:
                pltpu.make_async_copy(ones_v, acc_sh.at[dstb[nb]],
                                      ssems[nb]).wait()
            kn = jnp.minimum(k + 1, NCHUNKS - 1)
            pltpu.async_copy(idx_hbm.at[wid, kn, 1], dstb[nb], dsems[nb])
            pltpu.make_async_copy(idx_hbm.at[wid, 0, 1], dstb[b],
                                  dsems[b]).wait()
            pltpu.async_copy(ones_v, acc_sh.at[dstb[b]], ssems[b], add=True)
        return 0

    lax.fori_loop(0, NCHUNKS // 2, pair, 0)
    pltpu.make_async_copy(ones_v, acc_sh.at[dstb[1]], ssems[1]).wait()
    pltpu.make_async_copy(idx_hbm.at[wid, 0, 1], dstb[0], dsems[0]).wait()
    plsc.subcore_barrier()
    _copy_out(acc_sh, out_hbm, c, s)


ROWS_BLK = 2000


def _invdeg_body(dp_ref, o_ref):
    deg = dp_ref[0, :, :1] + dp_ref[1, :, :1]
    o_ref[...] = jnp.broadcast_to(1.0 / jnp.maximum(deg, 1.0), (ROWS_BLK, D))


def _tc_invdeg(deg_p):
    return pl.pallas_call(
        _invdeg_body,
        grid=(N // ROWS_BLK,),
        in_specs=[pl.BlockSpec((NC, ROWS_BLK, DEG_W), lambda i: (0, i, 0))],
        out_specs=pl.BlockSpec((ROWS_BLK, D), lambda i: (i, 0)),
        out_shape=jax.ShapeDtypeStruct((N, D), jnp.float32),
    )(deg_p)


def _layer_body(relu, p_ref, h_ref, inv_ref, wl_ref, wr_ref, b_ref, o_ref):
    agg = (p_ref[0] + p_ref[1]) * inv_ref[...]
    dn = (((1,), (1,)), ((), ()))
    acc = lax.dot_general(agg, wl_ref[...], dn, preferred_element_type=jnp.float32)
    acc = acc + lax.dot_general(h_ref[...], wr_ref[...], dn,
                                preferred_element_type=jnp.float32)
    acc = acc + b_ref[...]
    o_ref[...] = jnp.maximum(acc, 0.0) if relu else acc


def _tc_layer(p, hp, invd, wl, wr, bb, relu):
    return pl.pallas_call(
        functools.partial(_layer_body, relu),
        grid=(N // ROWS_BLK,),
        in_specs=[
            pl.BlockSpec((NC, ROWS_BLK, D), lambda i: (0, i, 0)),
            pl.BlockSpec((ROWS_BLK, D), lambda i: (i, 0)),
            pl.BlockSpec((ROWS_BLK, D), lambda i: (i, 0)),
            pl.BlockSpec((D, D), lambda i: (0, 0)),
            pl.BlockSpec((D, D), lambda i: (0, 0)),
            pl.BlockSpec((1, D), lambda i: (0, 0)),
        ],
        out_specs=pl.BlockSpec((ROWS_BLK, D), lambda i: (i, 0)),
        out_shape=jax.ShapeDtypeStruct((N_PAD, D), jnp.float32),
    )(p, hp, invd, wl, wr, bb)


def kernel(x, edge_index, Wl, Wr, b):
    src = edge_index[0].astype(jnp.int32)
    dst = edge_index[1].astype(jnp.int32)
    pad = E_PAD - E
    src_p = jnp.concatenate([src, jnp.full((pad,), N, jnp.int32)])
    dst_p = jnp.concatenate([dst, jnp.full((pad,), N, jnp.int32)])

    order = jnp.argsort(src_p)
    ss = src_p[order].reshape(TOTC, CHUNK)
    dd = dst_p[order].reshape(TOTC, CHUNK)
    first = ss[:, 0]
    last = ss[:, -1]
    base = jnp.minimum(first, N_PAD - SPAN).astype(jnp.int32)
    base = base - base % 8
    ok = (last - base) < SPAN
    src_local = jnp.clip(ss - base[:, None], 0, SPAN - 1).astype(jnp.int32)

    okt = ok.reshape(NW, NCHUNKS)
    perm = jnp.argsort(~okt, axis=1)
    fast_cnt = okt.sum(axis=1).astype(jnp.int32)
    fast_cnt = fast_cnt - fast_cnt % 2

    gidx = jnp.stack([ss, dd], axis=1).reshape(NW, NCHUNKS, 2, CHUNK)
    gidx = jnp.take_along_axis(gidx, perm[:, :, None, None], axis=1)
    is_start = jnp.concatenate(
        [jnp.ones((TOTC, 1), bool), src_local[:, 1:] != src_local[:, :-1]],
        axis=1)
    rid = jnp.cumsum(is_start, axis=1) - 1
    nruns = (rid[:, -1] + 1).astype(jnp.int32)
    rows_ix = jnp.arange(TOTC)[:, None]
    runrow = jnp.zeros((TOTC, 96), jnp.int32).at[rows_ix, rid].set(src_local)
    runlen = jnp.zeros((TOTC, 96), jnp.int32).at[rows_ix, rid].add(1)
    rmeta = jnp.stack([runrow, runlen], axis=1).reshape(NW, NCHUNKS, 2, 96)
    rmeta = jnp.take_along_axis(rmeta, perm[:, :, None, None], axis=1)
    nruns_t = jnp.take_along_axis(nruns.reshape(NW, NCHUNKS), perm, axis=1)
    base_t = jnp.take_along_axis(base.reshape(NW, NCHUNKS), perm, axis=1)

    deg_p = _sc_deg(gidx)
    invd = _tc_invdeg(deg_p)

    hp = jnp.concatenate([x, jnp.zeros((N_PAD - N, D), jnp.float32)])
    for i in range(L):
        p = _sc_agg(hp, gidx, rmeta, base_t, nruns_t, fast_cnt)
        hp = _tc_layer(p, hp, invd, Wl[i], Wr[i], b[i][None, :],
                       relu=(i < L - 1))
    return hp[:N]


# speedup vs baseline: 1.8773x; 1.0106x over previous
import functools

import jax
import jax.numpy as jnp
from jax import lax
from jax.experimental import pallas as pl
from jax.experimental.pallas import tpu as pltpu
from jax.experimental.pallas import tpu_sc as plsc

N = 10000
E = 320000
D = 128
L = 10

NC = 2
NS = 16
NW = NC * NS
CHUNK = 80
NCHUNKS = 128
TOTC = NW * NCHUNKS
EPT = CHUNK * NCHUNKS
E_PAD = EPT * NW
N_PAD = 10240
SLAB = N_PAD // NS
LAST = N - (NS - 1) * SLAB
SPAN = 16
DEG_W = D

_MESH = plsc.VectorSubcoreMesh(
    core_axis_name="c", subcore_axis_name="s", num_cores=NC, num_subcores=NS
)


def _fill(buf, val, width=D):
    vec = jnp.full((16,), val, jnp.float32)

    def body(r, _):
        for k in range(width // 16):
            buf[r, pl.ds(k * 16, 16)] = vec
        return 0

    lax.fori_loop(0, CHUNK, body, 0)


def _zero_slab(zbuf, acc_sh, s, width=D):
    _fill(zbuf, 0.0, width)
    slab = pl.multiple_of(s * SLAB, CHUNK)
    for k in range(SLAB // CHUNK):
        pltpu.sync_copy(zbuf, acc_sh.at[pl.ds(slab + k * CHUNK, CHUNK)])


def _copy_out(acc_sh, out_hbm, c, s):
    start = pl.multiple_of(s * SLAB, CHUNK)

    @pl.when(s < NS - 1)
    def _():
        pltpu.sync_copy(acc_sh.at[pl.ds(start, SLAB)],
                        out_hbm.at[c, pl.ds(start, SLAB)])

    @pl.when(s == NS - 1)
    def _():
        pltpu.sync_copy(acc_sh.at[pl.ds(start, LAST)],
                        out_hbm.at[c, pl.ds(start, LAST)])


@functools.partial(
    pl.kernel,
    out_type=jax.ShapeDtypeStruct((NC, N, D), jnp.float32),
    mesh=_MESH,
    scratch_types=[
        [pltpu.VMEM((SPAN, D), jnp.float32)] * 2,
        [pltpu.VMEM((CHUNK, D), jnp.float32)] * 2,
        [pltpu.VMEM((2, 96), jnp.int32)] * 2,
        [pltpu.VMEM((CHUNK,), jnp.int32)] * 2,
        pltpu.VMEM((NCHUNKS + 16,), jnp.int32),
        pltpu.VMEM((NCHUNKS + 16,), jnp.int32),
        pltpu.VMEM((32,), jnp.int32),
        pltpu.VMEM((CHUNK,), jnp.int32),
        pltpu.VMEM_SHARED((N_PAD, D), jnp.float32),
        [pltpu.SemaphoreType.DMA] * 2,
        [pltpu.SemaphoreType.DMA] * 2,
        [pltpu.SemaphoreType.DMA] * 2,
        [pltpu.SemaphoreType.DMA] * 2,
        pltpu.SemaphoreType.DMA,
    ],
)
def _sc_agg(h_hbm, gidx_hbm, rmeta_hbm, base_hbm, nruns_hbm, cnt_hbm, out_hbm,
            span, msg, rm, dstb, base_v, nruns_v, cnt_v, fbsrc, acc_sh,
            psems, isems, dsems, ssems, fsem):
    c = lax.axis_index("c")
    s = lax.axis_index("s")
    wid = s * NC + c

    pltpu.sync_copy(base_hbm.at[wid], base_v.at[pl.ds(0, NCHUNKS)])
    pltpu.sync_copy(nruns_hbm.at[wid], nruns_v.at[pl.ds(0, NCHUNKS)])
    pltpu.sync_copy(cnt_hbm.at[pl.ds((wid // 16) * 16, 16)],
                    cnt_v.at[pl.ds(0, 16)])
    fast_cnt = cnt_v[pl.ds(wid % 16, 16)][0]

    def chunk_base(k):
        return pl.multiple_of(base_v[pl.ds(k, 16)][0], 8)

    _zero_slab(msg[0], acc_sh, s)
    b0 = chunk_base(0)
    pltpu.async_copy(h_hbm.at[pl.ds(b0, SPAN)], span[0], psems[0])
    pltpu.async_copy(rmeta_hbm.at[wid, 0], rm[0], isems[0])
    pltpu.async_copy(gidx_hbm.at[wid, 0, 1], dstb[0], dsems[0])
    plsc.subcore_barrier()

    cols = [lax.iota(jnp.int32, 16) + 16 * kk for kk in range(D // 16)]

    def fast_pair(g, _):
        for b in range(2):
            k = 2 * g + b
            nb = 1 - b
            if b == 0:
                @pl.when(g >= 1)
                def _():
                    pltpu.make_async_copy(msg[nb], acc_sh.at[dstb[nb]],
                                          ssems[nb]).wait()
            else:
                pltpu.make_async_copy(msg[nb], acc_sh.at[dstb[nb]],
                                      ssems[nb]).wait()
            kn = jnp.minimum(k + 1, NCHUNKS - 1)
            bn = chunk_base(kn)
            pltpu.async_copy(h_hbm.at[pl.ds(bn, SPAN)], span[nb], psems[nb])
            pltpu.async_copy(rmeta_hbm.at[wid, kn], rm[nb], isems[nb])
            pltpu.async_copy(gidx_hbm.at[wid, kn, 1], dstb[nb], dsems[nb])
            pltpu.make_async_copy(h_hbm.at[pl.ds(bn, SPAN)], span[b],
                                  psems[b]).wait()
            pltpu.make_async_copy(rmeta_hbm.at[wid, 0], rm[b],
                                  isems[b]).wait()
            pltpu.make_async_copy(gidx_hbm.at[wid, 0, 1], dstb[b],
                                  dsems[b]).wait()

            nr = nruns_v[pl.ds(k, 16)][0]

            def run(j, pos):
                r = rm[b][0, pl.ds(j, 16)][0]
                ln = rm[b][1, pl.ds(j, 16)][0]
                vs = [span[b][r, pl.ds(16 * kk, 16)] for kk in range(D // 16)]

                def put(i, _):
                    for kk in range(D // 16):
                        msg[b][pos + i, pl.ds(16 * kk, 16)] = vs[kk]
                    return 0

                lax.fori_loop(0, ln, put, 0)
                return pos + ln

            lax.fori_loop(0, nr, run, 0)
            pltpu.async_copy(msg[b], acc_sh.at[dstb[b]], ssems[b], add=True)
        return 0

    lax.fori_loop(0, lax.div(fast_cnt, 2), fast_pair, 0)

    @pl.when(fast_cnt > 0)
    def _():
        pltpu.make_async_copy(msg[1], acc_sh.at[dstb[1]], ssems[1]).wait()
        pltpu.make_async_copy(gidx_hbm.at[wid, 0, 1], dstb[0],
                              dsems[0]).wait()
    pltpu.make_async_copy(h_hbm.at[pl.ds(b0, SPAN)], span[0], psems[0]).wait()
    pltpu.make_async_copy(rmeta_hbm.at[wid, 0], rm[0], isems[0]).wait()

    @pl.when(fast_cnt == 0)
    def _():
        pltpu.make_async_copy(gidx_hbm.at[wid, 0, 1], dstb[0],
                              dsems[0]).wait()

    def fb_chunk(k, _):
        pltpu.sync_copy(gidx_hbm.at[wid, k, 0], fbsrc)
        pltpu.sync_copy(gidx_hbm.at[wid, k, 1], dstb[0])
        pltpu.async_copy(h_hbm.at[fbsrc], msg[0], fsem).wait()
        pltpu.sync_copy(msg[0], acc_sh.at[dstb[0]], add=True)
        return 0

    lax.fori_loop(fast_cnt, NCHUNKS, fb_chunk, 0)

    plsc.subcore_barrier()
    _copy_out(acc_sh, out_hbm, c, s)


@functools.partial(
    pl.kernel,
    out_type=jax.ShapeDtypeStruct((NC, N, DEG_W), jnp.float32),
    mesh=_MESH,
    scratch_types=[
        pltpu.VMEM((CHUNK, DEG_W), jnp.float32),
        [pltpu.VMEM((CHUNK,), jnp.int32)] * 2,
        pltpu.VMEM_SHARED((N_PAD, DEG_W), jnp.float32),
        [pltpu.SemaphoreType.DMA] * 2,
        [pltpu.SemaphoreType.DMA] * 2,
    ],
)
def _sc_deg(idx_hbm, out_hbm, ones_v, dstb, acc_sh, dsems, ssems):
    c = lax.axis_index("c")
    s = lax.axis_index("s")
    wid = s * NC + c

    pltpu.async_copy(idx_hbm.at[wid, 0, 1], dstb[0], dsems[0])
    _zero_slab(ones_v, acc_sh, s, DEG_W)
    _fill(ones_v, 1.0, DEG_W)
    plsc.subcore_barrier()

    def pair(g, _):
        for b in range(2):
            k = 2 * g + b
            nb = 1 - b
            if b == 0:
                @pl.when(g >= 1)
                def _():
                    pltpu.make_async_copy(ones_v, acc_sh.at[dstb[nb]],
                                          ssems[nb]).wait()
            else:
                pltpu.make_async_copy(ones_v, acc_sh.at[dstb[nb]],
                                      ssems[nb]).wait()
            kn = jnp.minimum(k + 1, NCHUNKS - 1)
            pltpu.async_copy(idx_hbm.at[wid, kn, 1], dstb[nb], dsems[nb])
            pltpu.make_async_copy(idx_hbm.at[wid, 0, 1], dstb[b],
                                  dsems[b]).wait()
            pltpu.async_copy(ones_v, acc_sh.at[dstb[b]], ssems[b], add=True)
        return 0

    lax.fori_loop(0, NCHUNKS // 2, pair, 0)
    pltpu.make_async_copy(ones_v, acc_sh.at[dstb[1]], ssems[1]).wait()
    pltpu.make_async_copy(idx_hbm.at[wid, 0, 1], dstb[0], dsems[0]).wait()
    plsc.subcore_barrier()
    _copy_out(acc_sh, out_hbm, c, s)


ROWS_BLK = 2000


def _invdeg_body(dp_ref, o_ref):
    deg = dp_ref[0, :, :1] + dp_ref[1, :, :1]
    o_ref[...] = jnp.broadcast_to(1.0 / jnp.maximum(deg, 1.0), (ROWS_BLK, D))


def _tc_invdeg(deg_p):
    return pl.pallas_call(
        _invdeg_body,
        grid=(N // ROWS_BLK,),
        in_specs=[pl.BlockSpec((NC, ROWS_BLK, DEG_W), lambda i: (0, i, 0))],
        out_specs=pl.BlockSpec((ROWS_BLK, D), lambda i: (i, 0)),
        out_shape=jax.ShapeDtypeStruct((N, D), jnp.float32),
    )(deg_p)


def _layer_body(relu, p_ref, h_ref, inv_ref, wl_ref, wr_ref, b_ref, o_ref):
    agg = (p_ref[0] + p_ref[1]) * inv_ref[...]
    dn = (((1,), (1,)), ((), ()))
    acc = lax.dot_general(agg, wl_ref[...], dn, preferred_element_type=jnp.float32)
    acc = acc + lax.dot_general(h_ref[...], wr_ref[...], dn,
                                preferred_element_type=jnp.float32)
    acc = acc + b_ref[...]
    o_ref[...] = jnp.maximum(acc, 0.0) if relu else acc


def _tc_layer(p, hp, invd, wl, wr, bb, relu):
    return pl.pallas_call(
        functools.partial(_layer_body, relu),
        grid=(N // ROWS_BLK,),
        in_specs=[
            pl.BlockSpec((NC, ROWS_BLK, D), lambda i: (0, i, 0)),
            pl.BlockSpec((ROWS_BLK, D), lambda i: (i, 0)),
            pl.BlockSpec((ROWS_BLK, D), lambda i: (i, 0)),
            pl.BlockSpec((D, D), lambda i: (0, 0)),
            pl.BlockSpec((D, D), lambda i: (0, 0)),
            pl.BlockSpec((1, D), lambda i: (0, 0)),
        ],
        out_specs=pl.BlockSpec((ROWS_BLK, D), lambda i: (i, 0)),
        out_shape=jax.ShapeDtypeStruct((N_PAD, D), jnp.float32),
    )(p, hp, invd, wl, wr, bb)


def kernel(x, edge_index, Wl, Wr, b):
    src = edge_index[0].astype(jnp.int32)
    dst = edge_index[1].astype(jnp.int32)
    pad = E_PAD - E
    src_p = jnp.concatenate([src, jnp.full((pad,), N, jnp.int32)])
    dst_p = jnp.concatenate([dst, jnp.full((pad,), N, jnp.int32)])

    order = jnp.argsort(src_p)
    ss = src_p[order].reshape(TOTC, CHUNK)
    dd = dst_p[order].reshape(TOTC, CHUNK)
    first = ss[:, 0]
    last = ss[:, -1]
    base = jnp.minimum(first, N_PAD - SPAN).astype(jnp.int32)
    base = base - base % 8
    ok = (last - base) < SPAN
    src_local = jnp.clip(ss - base[:, None], 0, SPAN - 1).astype(jnp.int32)

    okt = ok.reshape(NW, NCHUNKS)
    perm = jnp.argsort(~okt, axis=1)
    fast_cnt = okt.sum(axis=1).astype(jnp.int32)
    fast_cnt = fast_cnt - fast_cnt % 2

    gidx = jnp.stack([ss, dd], axis=1).reshape(NW, NCHUNKS, 2, CHUNK)
    gidx = jnp.take_along_axis(gidx, perm[:, :, None, None], axis=1)
    is_start = jnp.concatenate(
        [jnp.ones((TOTC, 1), bool), src_local[:, 1:] != src_local[:, :-1]],
        axis=1)
    rid = jnp.cumsum(is_start, axis=1) - 1
    nruns = (rid[:, -1] + 1).astype(jnp.int32)
    rows_ix = jnp.arange(TOTC)[:, None]
    runrow = jnp.zeros((TOTC, 96), jnp.int32).at[rows_ix, rid].set(src_local)
    runlen = jnp.zeros((TOTC, 96), jnp.int32).at[rows_ix, rid].add(1)
    rmeta = jnp.stack([runrow, runlen], axis=1).reshape(NW, NCHUNKS, 2, 96)
    rmeta = jnp.take_along_axis(rmeta, perm[:, :, None, None], axis=1)
    nruns_t = jnp.take_along_axis(nruns.reshape(NW, NCHUNKS), perm, axis=1)
    base_t = jnp.take_along_axis(base.reshape(NW, NCHUNKS), perm, axis=1)

    deg_p = _sc_deg(gidx)
    invd = _tc_invdeg(deg_p)

    hp = jnp.concatenate([x, jnp.zeros((N_PAD - N, D), jnp.float32)])
    for i in range(L):
        p = _sc_agg(hp, gidx, rmeta, base_t, nruns_t, fast_cnt)
        hp = _tc_layer(p, hp, invd, Wl[i], Wr[i], b[i][None, :],
                       relu=(i < L - 1))
    return hp[:N]


# speedup vs baseline: 1.8814x; 1.0021x over previous
import functools

import jax
import jax.numpy as jnp
from jax import lax
from jax.experimental import pallas as pl
from jax.experimental.pallas import tpu as pltpu
from jax.experimental.pallas import tpu_sc as plsc

N = 10000
E = 320000
D = 128
L = 10

NC = 2
NS = 16
NW = NC * NS
CHUNK = 80
NCHUNKS = 128
TOTC = NW * NCHUNKS
EPT = CHUNK * NCHUNKS
E_PAD = EPT * NW
N_PAD = 10240
SLAB = N_PAD // NS
LAST = N - (NS - 1) * SLAB
SPAN = 16
DEG_W = D

_MESH = plsc.VectorSubcoreMesh(
    core_axis_name="c", subcore_axis_name="s", num_cores=NC, num_subcores=NS
)


def _fill(buf, val, width=D):
    vec = jnp.full((16,), val, jnp.float32)

    def body(r, _):
        for k in range(width // 16):
            buf[r, pl.ds(k * 16, 16)] = vec
        return 0

    lax.fori_loop(0, CHUNK, body, 0)


def _zero_slab(zbuf, acc_sh, s, width=D):
    _fill(zbuf, 0.0, width)
    slab = pl.multiple_of(s * SLAB, CHUNK)
    for k in range(SLAB // CHUNK):
        pltpu.sync_copy(zbuf, acc_sh.at[pl.ds(slab + k * CHUNK, CHUNK)])


def _copy_out(acc_sh, out_hbm, c, s):
    start = pl.multiple_of(s * SLAB, CHUNK)

    @pl.when(s < NS - 1)
    def _():
        pltpu.sync_copy(acc_sh.at[pl.ds(start, SLAB)],
                        out_hbm.at[c, pl.ds(start, SLAB)])

    @pl.when(s == NS - 1)
    def _():
        pltpu.sync_copy(acc_sh.at[pl.ds(start, LAST)],
                        out_hbm.at[c, pl.ds(start, LAST)])


@functools.partial(
    pl.kernel,
    out_type=jax.ShapeDtypeStruct((NC, N, D), jnp.float32),
    mesh=_MESH,
    scratch_types=[
        [pltpu.VMEM((SPAN, D), jnp.float32)] * 2,
        [pltpu.VMEM((CHUNK, D), jnp.float32)] * 2,
        [pltpu.VMEM((2, 96), jnp.int32)] * 2,
        [pltpu.VMEM((CHUNK,), jnp.int32)] * 2,
        pltpu.VMEM((NCHUNKS + 16,), jnp.int32),
        pltpu.VMEM((NCHUNKS + 16,), jnp.int32),
        pltpu.VMEM((32,), jnp.int32),
        pltpu.VMEM((CHUNK,), jnp.int32),
        pltpu.VMEM_SHARED((N_PAD, D), jnp.float32),
        [pltpu.SemaphoreType.DMA] * 2,
        [pltpu.SemaphoreType.DMA] * 2,
        [pltpu.SemaphoreType.DMA] * 2,
        [pltpu.SemaphoreType.DMA] * 2,
        pltpu.SemaphoreType.DMA,
    ],
)
def _sc_agg(h_hbm, gidx_hbm, rmeta_hbm, base_hbm, nruns_hbm, cnt_hbm, out_hbm,
            span, msg, rm, dstb, base_v, nruns_v, cnt_v, fbsrc, acc_sh,
            psems, isems, dsems, ssems, fsem):
    c = lax.axis_index("c")
    s = lax.axis_index("s")
    wid = s * NC + c

    pltpu.sync_copy(base_hbm.at[wid], base_v.at[pl.ds(0, NCHUNKS)])
    pltpu.sync_copy(nruns_hbm.at[wid], nruns_v.at[pl.ds(0, NCHUNKS)])
    pltpu.sync_copy(cnt_hbm.at[pl.ds((wid // 16) * 16, 16)],
                    cnt_v.at[pl.ds(0, 16)])
    fast_cnt = cnt_v[pl.ds(wid % 16, 16)][0]

    def chunk_base(k):
        return pl.multiple_of(base_v[pl.ds(k, 16)][0], 8)

    _zero_slab(msg[0], acc_sh, s)
    b0 = chunk_base(0)
    pltpu.async_copy(h_hbm.at[pl.ds(b0, SPAN)], span[0], psems[0])
    pltpu.async_copy(rmeta_hbm.at[wid, 0], rm[0], isems[0])
    pltpu.async_copy(gidx_hbm.at[wid, 0, 1], dstb[0], dsems[0])
    plsc.subcore_barrier()

    cols = [lax.iota(jnp.int32, 16) + 16 * kk for kk in range(D // 16)]

    def fast_pair(g, _):
        for b in range(2):
            k = 2 * g + b
            nb = 1 - b
            if b == 0:
                @pl.when(g >= 1)
                def _():
                    pltpu.make_async_copy(msg[nb], acc_sh.at[dstb[nb]],
                                          ssems[nb]).wait()
            else:
                pltpu.make_async_copy(msg[nb], acc_sh.at[dstb[nb]],
                                      ssems[nb]).wait()
            kn = jnp.minimum(k + 1, NCHUNKS - 1)
            bn = chunk_base(kn)
            pltpu.async_copy(h_hbm.at[pl.ds(bn, SPAN)], span[nb], psems[nb])
            pltpu.async_copy(rmeta_hbm.at[wid, kn], rm[nb], isems[nb])
            pltpu.async_copy(gidx_hbm.at[wid, kn, 1], dstb[nb], dsems[nb])
            pltpu.make_async_copy(h_hbm.at[pl.ds(bn, SPAN)], span[b],
                                  psems[b]).wait()
            pltpu.make_async_copy(rmeta_hbm.at[wid, 0], rm[b],
                                  isems[b]).wait()
            pltpu.make_async_copy(gidx_hbm.at[wid, 0, 1], dstb[b],
                                  dsems[b]).wait()

            nr = nruns_v[pl.ds(k, 16)][0]

            def run(j, pos):
                r = rm[b][0, pl.ds(j, 16)][0]
                ln = rm[b][1, pl.ds(j, 16)][0]
                vs = [span[b][r, pl.ds(16 * kk, 16)] for kk in range(D // 16)]

                def put2(i, _):
                    for kk in range(D // 16):
                        msg[b][pos + 2 * i, pl.ds(16 * kk, 16)] = vs[kk]
                        msg[b][pos + 2 * i + 1, pl.ds(16 * kk, 16)] = vs[kk]
                    return 0

                def put(i, _):
                    for kk in range(D // 16):
                        msg[b][pos + i, pl.ds(16 * kk, 16)] = vs[kk]
                    return 0

                lax.fori_loop(0, ln // 2, put2, 0)
                lax.fori_loop(2 * (ln // 2), ln, put, 0)
                return pos + ln

            lax.fori_loop(0, nr, run, 0)
            pltpu.async_copy(msg[b], acc_sh.at[dstb[b]], ssems[b], add=True)
        return 0

    lax.fori_loop(0, lax.div(fast_cnt, 2), fast_pair, 0)

    @pl.when(fast_cnt > 0)
    def _():
        pltpu.make_async_copy(msg[1], acc_sh.at[dstb[1]], ssems[1]).wait()
        pltpu.make_async_copy(gidx_hbm.at[wid, 0, 1], dstb[0],
                              dsems[0]).wait()
    pltpu.make_async_copy(h_hbm.at[pl.ds(b0, SPAN)], span[0], psems[0]).wait()
    pltpu.make_async_copy(rmeta_hbm.at[wid, 0], rm[0], isems[0]).wait()

    @pl.when(fast_cnt == 0)
    def _():
        pltpu.make_async_copy(gidx_hbm.at[wid, 0, 1], dstb[0],
                              dsems[0]).wait()

    def fb_chunk(k, _):
        pltpu.sync_copy(gidx_hbm.at[wid, k, 0], fbsrc)
        pltpu.sync_copy(gidx_hbm.at[wid, k, 1], dstb[0])
        pltpu.async_copy(h_hbm.at[fbsrc], msg[0], fsem).wait()
        pltpu.sync_copy(msg[0], acc_sh.at[dstb[0]], add=True)
        return 0

    lax.fori_loop(fast_cnt, NCHUNKS, fb_chunk, 0)

    plsc.subcore_barrier()
    _copy_out(acc_sh, out_hbm, c, s)


@functools.partial(
    pl.kernel,
    out_type=jax.ShapeDtypeStruct((NC, N, DEG_W), jnp.float32),
    mesh=_MESH,
    scratch_types=[
        pltpu.VMEM((CHUNK, DEG_W), jnp.float32),
        [pltpu.VMEM((CHUNK,), jnp.int32)] * 2,
        pltpu.VMEM_SHARED((N_PAD, DEG_W), jnp.float32),
        [pltpu.SemaphoreType.DMA] * 2,
        [pltpu.SemaphoreType.DMA] * 2,
    ],
)
def _sc_deg(idx_hbm, out_hbm, ones_v, dstb, acc_sh, dsems, ssems):
    c = lax.axis_index("c")
    s = lax.axis_index("s")
    wid = s * NC + c

    pltpu.async_copy(idx_hbm.at[wid, 0, 1], dstb[0], dsems[0])
    _zero_slab(ones_v, acc_sh, s, DEG_W)
    _fill(ones_v, 1.0, DEG_W)
    plsc.subcore_barrier()

    def pair(g, _):
        for b in range(2):
            k = 2 * g + b
            nb = 1 - b
            if b == 0:
                @pl.when(g >= 1)
                def _():
                    pltpu.make_async_copy(ones_v, acc_sh.at[dstb[nb]],
                                          ssems[nb]).wait()
            else:
                pltpu.make_async_copy(ones_v, acc_sh.at[dstb[nb]],
                                      ssems[nb]).wait()
            kn = jnp.minimum(k + 1, NCHUNKS - 1)
            pltpu.async_copy(idx_hbm.at[wid, kn, 1], dstb[nb], dsems[nb])
            pltpu.make_async_copy(idx_hbm.at[wid, 0, 1], dstb[b],
                                  dsems[b]).wait()
            pltpu.async_copy(ones_v, acc_sh.at[dstb[b]], ssems[b], add=True)
        return 0

    lax.fori_loop(0, NCHUNKS // 2, pair, 0)
    pltpu.make_async_copy(ones_v, acc_sh.at[dstb[1]], ssems[1]).wait()
    pltpu.make_async_copy(idx_hbm.at[wid, 0, 1], dstb[0], dsems[0]).wait()
    plsc.subcore_barrier()
    _copy_out(acc_sh, out_hbm, c, s)


ROWS_BLK = 2000


def _invdeg_body(dp_ref, o_ref):
    deg = dp_ref[0, :, :1] + dp_ref[1, :, :1]
    o_ref[...] = jnp.broadcast_to(1.0 / jnp.maximum(deg, 1.0), (ROWS_BLK, D))


def _tc_invdeg(deg_p):
    return pl.pallas_call(
        _invdeg_body,
        grid=(N // ROWS_BLK,),
        in_specs=[pl.BlockSpec((NC, ROWS_BLK, DEG_W), lambda i: (0, i, 0))],
        out_specs=pl.BlockSpec((ROWS_BLK, D), lambda i: (i, 0)),
        out_shape=jax.ShapeDtypeStruct((N, D), jnp.float32),
    )(deg_p)


def _layer_body(relu, p_ref, h_ref, inv_ref, wl_ref, wr_ref, b_ref, o_ref):
    agg = (p_ref[0] + p_ref[1]) * inv_ref[...]
    dn = (((1,), (1,)), ((), ()))
    acc = lax.dot_general(agg, wl_ref[...], dn, preferred_element_type=jnp.float32)
    acc = acc + lax.dot_general(h_ref[...], wr_ref[...], dn,
                                preferred_element_type=jnp.float32)
    acc = acc + b_ref[...]
    o_ref[...] = jnp.maximum(acc, 0.0) if relu else acc


def _tc_layer(p, hp, invd, wl, wr, bb, relu):
    return pl.pallas_call(
        functools.partial(_layer_body, relu),
        grid=(N // ROWS_BLK,),
        in_specs=[
            pl.BlockSpec((NC, ROWS_BLK, D), lambda i: (0, i, 0)),
            pl.BlockSpec((ROWS_BLK, D), lambda i: (i, 0)),
            pl.BlockSpec((ROWS_BLK, D), lambda i: (i, 0)),
            pl.BlockSpec((D, D), lambda i: (0, 0)),
            pl.BlockSpec((D, D), lambda i: (0, 0)),
            pl.BlockSpec((1, D), lambda i: (0, 0)),
        ],
        out_specs=pl.BlockSpec((ROWS_BLK, D), lambda i: (i, 0)),
        out_shape=jax.ShapeDtypeStruct((N_PAD, D), jnp.float32),
    )(p, hp, invd, wl, wr, bb)


def kernel(x, edge_index, Wl, Wr, b):
    src = edge_index[0].astype(jnp.int32)
    dst = edge_index[1].astype(jnp.int32)
    pad = E_PAD - E
    src_p = jnp.concatenate([src, jnp.full((pad,), N, jnp.int32)])
    dst_p = jnp.concatenate([dst, jnp.full((pad,), N, jnp.int32)])

    order = jnp.argsort(src_p)
    ss = src_p[order].reshape(TOTC, CHUNK)
    dd = dst_p[order].reshape(TOTC, CHUNK)
    first = ss[:, 0]
    last = ss[:, -1]
    base = jnp.minimum(first, N_PAD - SPAN).astype(jnp.int32)
    base = base - base % 8
    ok = (last - base) < SPAN
    src_local = jnp.clip(ss - base[:, None], 0, SPAN - 1).astype(jnp.int32)

    okt = ok.reshape(NW, NCHUNKS)
    perm = jnp.argsort(~okt, axis=1)
    fast_cnt = okt.sum(axis=1).astype(jnp.int32)
    fast_cnt = fast_cnt - fast_cnt % 2

    gidx = jnp.stack([ss, dd], axis=1).reshape(NW, NCHUNKS, 2, CHUNK)
    gidx = jnp.take_along_axis(gidx, perm[:, :, None, None], axis=1)
    is_start = jnp.concatenate(
        [jnp.ones((TOTC, 1), bool), src_local[:, 1:] != src_local[:, :-1]],
        axis=1)
    rid = jnp.cumsum(is_start, axis=1) - 1
    nruns = (rid[:, -1] + 1).astype(jnp.int32)
    rows_ix = jnp.arange(TOTC)[:, None]
    runrow = jnp.zeros((TOTC, 96), jnp.int32).at[rows_ix, rid].set(src_local)
    runlen = jnp.zeros((TOTC, 96), jnp.int32).at[rows_ix, rid].add(1)
    rmeta = jnp.stack([runrow, runlen], axis=1).reshape(NW, NCHUNKS, 2, 96)
    rmeta = jnp.take_along_axis(rmeta, perm[:, :, None, None], axis=1)
    nruns_t = jnp.take_along_axis(nruns.reshape(NW, NCHUNKS), perm, axis=1)
    base_t = jnp.take_along_axis(base.reshape(NW, NCHUNKS), perm, axis=1)

    deg_p = _sc_deg(gidx)
    invd = _tc_invdeg(deg_p)

    hp = jnp.concatenate([x, jnp.zeros((N_PAD - N, D), jnp.float32)])
    for i in range(L):
        p = _sc_agg(hp, gidx, rmeta, base_t, nruns_t, fast_cnt)
        hp = _tc_layer(p, hp, invd, Wl[i], Wr[i], b[i][None, :],
                       relu=(i < L - 1))
    return hp[:N]


# speedup vs baseline: 1.8871x; 1.0030x over previous
import functools

import jax
import jax.numpy as jnp
from jax import lax
from jax.experimental import pallas as pl
from jax.experimental.pallas import tpu as pltpu
from jax.experimental.pallas import tpu_sc as plsc

N = 10000
E = 320000
D = 128
L = 10

NC = 2
NS = 16
NW = NC * NS
CHUNK = 80
NCHUNKS = 128
TOTC = NW * NCHUNKS
EPT = CHUNK * NCHUNKS
E_PAD = EPT * NW
N_PAD = 10240
SLAB = N_PAD // NS
LAST = N - (NS - 1) * SLAB
SPAN = 16
DEG_W = D

_MESH = plsc.VectorSubcoreMesh(
    core_axis_name="c", subcore_axis_name="s", num_cores=NC, num_subcores=NS
)


def _fill(buf, val, width=D):
    vec = jnp.full((16,), val, jnp.float32)

    def body(r, _):
        for k in range(width // 16):
            buf[r, pl.ds(k * 16, 16)] = vec
        return 0

    lax.fori_loop(0, CHUNK, body, 0)


def _zero_slab(zbuf, acc_sh, s, width=D):
    _fill(zbuf, 0.0, width)
    slab = pl.multiple_of(s * SLAB, CHUNK)
    for k in range(SLAB // CHUNK):
        pltpu.sync_copy(zbuf, acc_sh.at[pl.ds(slab + k * CHUNK, CHUNK)])


def _copy_out(acc_sh, out_hbm, c, s):
    start = pl.multiple_of(s * SLAB, CHUNK)

    @pl.when(s < NS - 1)
    def _():
        pltpu.sync_copy(acc_sh.at[pl.ds(start, SLAB)],
                        out_hbm.at[c, pl.ds(start, SLAB)])

    @pl.when(s == NS - 1)
    def _():
        pltpu.sync_copy(acc_sh.at[pl.ds(start, LAST)],
                        out_hbm.at[c, pl.ds(start, LAST)])


@functools.partial(
    pl.kernel,
    out_type=jax.ShapeDtypeStruct((NC, N, D), jnp.float32),
    mesh=_MESH,
    scratch_types=[
        [pltpu.VMEM((SPAN, D), jnp.float32)] * 2,
        [pltpu.VMEM((CHUNK, D), jnp.float32)] * 2,
        [pltpu.VMEM((2, 96), jnp.int32)] * 2,
        [pltpu.VMEM((CHUNK,), jnp.int32)] * 2,
        pltpu.VMEM((NCHUNKS + 16,), jnp.int32),
        pltpu.VMEM((NCHUNKS + 16,), jnp.int32),
        pltpu.VMEM((32,), jnp.int32),
        pltpu.VMEM((CHUNK,), jnp.int32),
        pltpu.VMEM_SHARED((N_PAD, D), jnp.float32),
        [pltpu.SemaphoreType.DMA] * 2,
        [pltpu.SemaphoreType.DMA] * 2,
        [pltpu.SemaphoreType.DMA] * 2,
        [pltpu.SemaphoreType.DMA] * 2,
        pltpu.SemaphoreType.DMA,
    ],
)
def _sc_agg(h_hbm, gidx_hbm, rmeta_hbm, base_hbm, nruns_hbm, cnt_hbm, out_hbm,
            span, msg, rm, dstb, base_v, nruns_v, cnt_v, fbsrc, acc_sh,
            psems, isems, dsems, ssems, fsem):
    c = lax.axis_index("c")
    s = lax.axis_index("s")
    wid = s * NC + c

    pltpu.sync_copy(base_hbm.at[wid], base_v.at[pl.ds(0, NCHUNKS)])
    pltpu.sync_copy(nruns_hbm.at[wid], nruns_v.at[pl.ds(0, NCHUNKS)])
    pltpu.sync_copy(cnt_hbm.at[pl.ds((wid // 16) * 16, 16)],
                    cnt_v.at[pl.ds(0, 16)])
    fast_cnt = cnt_v[pl.ds(wid % 16, 16)][0]

    def chunk_base(k):
        return pl.multiple_of(base_v[pl.ds(k, 16)][0], 8)

    _zero_slab(msg[0], acc_sh, s)
    b0 = chunk_base(0)
    pltpu.async_copy(h_hbm.at[pl.ds(b0, SPAN)], span[0], psems[0])
    pltpu.async_copy(rmeta_hbm.at[wid, 0], rm[0], isems[0])
    pltpu.async_copy(gidx_hbm.at[wid, 0, 1], dstb[0], dsems[0])
    plsc.subcore_barrier()

    cols = [lax.iota(jnp.int32, 16) + 16 * kk for kk in range(D // 16)]

    def fast_pair(g, _):
        for b in range(2):
            k = 2 * g + b
            nb = 1 - b
            if b == 0:
                @pl.when(g >= 1)
                def _():
                    pltpu.make_async_copy(msg[nb], acc_sh.at[dstb[nb]],
                                          ssems[nb]).wait()
            else:
                pltpu.make_async_copy(msg[nb], acc_sh.at[dstb[nb]],
                                      ssems[nb]).wait()
            kn = jnp.minimum(k + 1, NCHUNKS - 1)
            bn = chunk_base(kn)
            pltpu.async_copy(h_hbm.at[pl.ds(bn, SPAN)], span[nb], psems[nb])
            pltpu.async_copy(rmeta_hbm.at[wid, kn], rm[nb], isems[nb])
            pltpu.async_copy(gidx_hbm.at[wid, kn, 1], dstb[nb], dsems[nb])
            pltpu.make_async_copy(h_hbm.at[pl.ds(bn, SPAN)], span[b],
                                  psems[b]).wait()
            pltpu.make_async_copy(rmeta_hbm.at[wid, 0], rm[b],
                                  isems[b]).wait()
            pltpu.make_async_copy(gidx_hbm.at[wid, 0, 1], dstb[b],
                                  dsems[b]).wait()

            nr = nruns_v[pl.ds(k, 16)][0]

            def run(j, pos):
                r = rm[b][0, pl.ds(j, 16)][0]
                ln = rm[b][1, pl.ds(j, 16)][0]
                vs = [span[b][r, pl.ds(16 * kk, 16)] for kk in range(D // 16)]

                def put(i, _):
                    for kk in range(D // 16):
                        msg[b][pos + i, pl.ds(16 * kk, 16)] = vs[kk]
                    return 0

                lax.fori_loop(0, ln, put, 0)
                return pos + ln

            lax.fori_loop(0, nr, run, 0)
            pltpu.async_copy(msg[b], acc_sh.at[dstb[b]], ssems[b], add=True)
        return 0

    lax.fori_loop(0, lax.div(fast_cnt, 2), fast_pair, 0)

    @pl.when(fast_cnt > 0)
    def _():
        pltpu.make_async_copy(msg[1], acc_sh.at[dstb[1]], ssems[1]).wait()
        pltpu.make_async_copy(gidx_hbm.at[wid, 0, 1], dstb[0],
                              dsems[0]).wait()
    pltpu.make_async_copy(h_hbm.at[pl.ds(b0, SPAN)], span[0], psems[0]).wait()
    pltpu.make_async_copy(rmeta_hbm.at[wid, 0], rm[0], isems[0]).wait()

    @pl.when(fast_cnt == 0)
    def _():
        pltpu.make_async_copy(gidx_hbm.at[wid, 0, 1], dstb[0],
                              dsems[0]).wait()

    def fb_chunk(k, _):
        pltpu.sync_copy(gidx_hbm.at[wid, k, 0], fbsrc)
        pltpu.sync_copy(gidx_hbm.at[wid, k, 1], dstb[0])
        pltpu.async_copy(h_hbm.at[fbsrc], msg[0], fsem).wait()
        pltpu.sync_copy(msg[0], acc_sh.at[dstb[0]], add=True)
        return 0

    lax.fori_loop(fast_cnt, NCHUNKS, fb_chunk, 0)

    plsc.subcore_barrier()
    _copy_out(acc_sh, out_hbm, c, s)


@functools.partial(
    pl.kernel,
    out_type=jax.ShapeDtypeStruct((NC, N, DEG_W), jnp.float32),
    mesh=_MESH,
    scratch_types=[
        pltpu.VMEM((CHUNK, DEG_W), jnp.float32),
        [pltpu.VMEM((CHUNK,), jnp.int32)] * 2,
        pltpu.VMEM_SHARED((N_PAD, DEG_W), jnp.float32),
        [pltpu.SemaphoreType.DMA] * 2,
        [pltpu.SemaphoreType.DMA] * 2,
    ],
)
def _sc_deg(idx_hbm, out_hbm, ones_v, dstb, acc_sh, dsems, ssems):
    c = lax.axis_index("c")
    s = lax.axis_index("s")
    wid = s * NC + c

    pltpu.async_copy(idx_hbm.at[wid, 0, 1], dstb[0], dsems[0])
    _zero_slab(ones_v, acc_sh, s, DEG_W)
    _fill(ones_v, 1.0, DEG_W)
    plsc.subcore_barrier()

    def pair(g, _):
        for b in range(2):
            k = 2 * g + b
            nb = 1 - b
            if b == 0:
                @pl.when(g >= 1)
                def _():
                    pltpu.make_async_copy(ones_v, acc_sh.at[dstb[nb]],
                                          ssems[nb]).wait()
            else:
                pltpu.make_async_copy(ones_v, acc_sh.at[dstb[nb]],
                                      ssems[nb]).wait()
            kn = jnp.minimum(k + 1, NCHUNKS - 1)
            pltpu.async_copy(idx_hbm.at[wid, kn, 1], dstb[nb], dsems[nb])
            pltpu.make_async_copy(idx_hbm.at[wid, 0, 1], dstb[b],
                                  dsems[b]).wait()
            pltpu.async_copy(ones_v, acc_sh.at[dstb[b]], ssems[b], add=True)
        return 0

    lax.fori_loop(0, NCHUNKS // 2, pair, 0)
    pltpu.make_async_copy(ones_v, acc_sh.at[dstb[1]], ssems[1]).wait()
    pltpu.make_async_copy(idx_hbm.at[wid, 0, 1], dstb[0], dsems[0]).wait()
    plsc.subcore_barrier()
    _copy_out(acc_sh, out_hbm, c, s)


ROWS_BLK = 2000


def _invdeg_body(dp_ref, o_ref):
    deg = dp_ref[0, :, :1] + dp_ref[1, :, :1]
    o_ref[...] = jnp.broadcast_to(1.0 / jnp.maximum(deg, 1.0), (ROWS_BLK, D))


def _tc_invdeg(deg_p):
    return pl.pallas_call(
        _invdeg_body,
        grid=(N // ROWS_BLK,),
        in_specs=[pl.BlockSpec((NC, ROWS_BLK, DEG_W), lambda i: (0, i, 0))],
        out_specs=pl.BlockSpec((ROWS_BLK, D), lambda i: (i, 0)),
        out_shape=jax.ShapeDtypeStruct((N, D), jnp.float32),
    )(deg_p)


def _layer_body(relu, p_ref, h_ref, inv_ref, wl_ref, wr_ref, b_ref, o_ref):
    agg = (p_ref[0] + p_ref[1]) * inv_ref[...]
    dn = (((1,), (1,)), ((), ()))
    acc = lax.dot_general(agg, wl_ref[...], dn, preferred_element_type=jnp.float32)
    acc = acc + lax.dot_general(h_ref[...], wr_ref[...], dn,
                                preferred_element_type=jnp.float32)
    acc = acc + b_ref[...]
    o_ref[...] = jnp.maximum(acc, 0.0) if relu else acc


def _tc_layer(p, hp, invd, wl, wr, bb, relu):
    return pl.pallas_call(
        functools.partial(_layer_body, relu),
        grid=(N // ROWS_BLK,),
        in_specs=[
            pl.BlockSpec((NC, ROWS_BLK, D), lambda i: (0, i, 0)),
            pl.BlockSpec((ROWS_BLK, D), lambda i: (i, 0)),
            pl.BlockSpec((ROWS_BLK, D), lambda i: (i, 0)),
            pl.BlockSpec((D, D), lambda i: (0, 0)),
            pl.BlockSpec((D, D), lambda i: (0, 0)),
            pl.BlockSpec((1, D), lambda i: (0, 0)),
        ],
        out_specs=pl.BlockSpec((ROWS_BLK, D), lambda i: (i, 0)),
        out_shape=jax.ShapeDtypeStruct((N_PAD, D), jnp.float32),
    )(p, hp, invd, wl, wr, bb)


def kernel(x, edge_index, Wl, Wr, b):
    src = edge_index[0].astype(jnp.int32)
    dst = edge_index[1].astype(jnp.int32)
    pad = E_PAD - E
    src_p = jnp.concatenate([src, jnp.full((pad,), N, jnp.int32)])
    dst_p = jnp.concatenate([dst, jnp.full((pad,), N, jnp.int32)])

    order = jnp.argsort(src_p)
    ss = src_p[order].reshape(TOTC, CHUNK)
    dd = dst_p[order].reshape(TOTC, CHUNK)
    first = ss[:, 0]
    last = ss[:, -1]
    base = jnp.minimum(first, N_PAD - SPAN).astype(jnp.int32)
    base = base - base % 8
    ok = (last - base) < SPAN
    src_local = jnp.clip(ss - base[:, None], 0, SPAN - 1).astype(jnp.int32)

    okt = ok.reshape(NW, NCHUNKS)
    perm = jnp.argsort(~okt, axis=1)
    fast_cnt = okt.sum(axis=1).astype(jnp.int32)
    fast_cnt = fast_cnt - fast_cnt % 2

    gidx = jnp.stack([ss, dd], axis=1).reshape(NW, NCHUNKS, 2, CHUNK)
    gidx = jnp.take_along_axis(gidx, perm[:, :, None, None], axis=1)
    is_start = jnp.concatenate(
        [jnp.ones((TOTC, 1), bool), src_local[:, 1:] != src_local[:, :-1]],
        axis=1)
    rid = jnp.cumsum(is_start, axis=1) - 1
    nruns = (rid[:, -1] + 1).astype(jnp.int32)
    rows_ix = jnp.arange(TOTC)[:, None]
    runrow = jnp.zeros((TOTC, 96), jnp.int32).at[rows_ix, rid].set(src_local)
    runlen = jnp.zeros((TOTC, 96), jnp.int32).at[rows_ix, rid].add(1)
    rmeta = jnp.stack([runrow, runlen], axis=1).reshape(NW, NCHUNKS, 2, 96)
    rmeta = jnp.take_along_axis(rmeta, perm[:, :, None, None], axis=1)
    nruns_t = jnp.take_along_axis(nruns.reshape(NW, NCHUNKS), perm, axis=1)
    base_t = jnp.take_along_axis(base.reshape(NW, NCHUNKS), perm, axis=1)

    deg_p = _sc_deg(gidx)
    invd = _tc_invdeg(deg_p)

    hp = jnp.concatenate([x, jnp.zeros((N_PAD - N, D), jnp.float32)])
    for i in range(L):
        p = _sc_agg(hp, gidx, rmeta, base_t, nruns_t, fast_cnt)
        hp = _tc_layer(p, hp, invd, Wl[i], Wr[i], b[i][None, :],
                       relu=(i < L - 1))
    return hp[:N]
